# baseline (speedup 1.0000x reference)
import functools

import jax
import jax.numpy as jnp
from jax import lax
from jax.experimental import pallas as pl
from jax.experimental.pallas import tpu as pltpu
from jax.experimental.pallas import tpu_sc as plsc

_N = 32768
_D = 768
_E = 64
_BN = 512

_NW = 32
_ROWS_PER_W = _N // _NW
_BLK = 128
_NBLK = _ROWS_PER_W // _BLK
_LANES = 16
_NG = _BLK // _LANES


def _tc_logits_body(x_ref, w1_ref, b1_ref, w2_ref, b2_ref, logits_ref):
    x = x_ref[...]
    mu = jnp.mean(x, axis=-1, keepdims=True)
    xc = x - mu
    var = jnp.mean(xc * xc, axis=-1, keepdims=True)
    xn = xc / jnp.sqrt(var + 1e-5)

    h = jnp.dot(xn, w1_ref[...], preferred_element_type=jnp.float32)
    h = h + b1_ref[...]
    h = 0.5 * h * (1.0 + jax.lax.erf(h * 0.7071067811865476))

    logits = jnp.dot(h, w2_ref[...], preferred_element_type=jnp.float32)
    logits_ref[...] = logits + b2_ref[...]


def _tc_logits(x, W1f, b1f, W2, b2f):
    grid = (_N // _BN,)
    return pl.pallas_call(
        _tc_logits_body,
        grid=grid,
        in_specs=[
            pl.BlockSpec((_BN, _D), lambda i: (i, 0)),
            pl.BlockSpec((_D, _D), lambda i: (0, 0)),
            pl.BlockSpec((1, _D), lambda i: (0, 0)),
            pl.BlockSpec((_D, _E), lambda i: (0, 0)),
            pl.BlockSpec((1, _E), lambda i: (0, 0)),
        ],
        out_specs=pl.BlockSpec((_BN, _E), lambda i: (i, 0)),
        out_shape=jax.ShapeDtypeStruct((_N, _E), jnp.float32),
        compiler_params=pltpu.CompilerParams(
            dimension_semantics=("arbitrary",),
        ),
    )(x, W1f, b1f, W2, b2f)


def _scan_top2_groups(in_tile, rows_list):
    ng = len(rows_list)
    neg = jnp.full((_LANES,), -3.4028235e38, dtype=jnp.float32)
    zero_i = jnp.zeros((_LANES,), dtype=jnp.int32)

    def body(e, carry):
        e_vec = jnp.full((_LANES,), e, dtype=jnp.int32)
        out = []
        for g in range(ng):
            l1, l2, i1, i2 = carry[g]
            v = plsc.load_gather(in_tile, [rows_list[g], e_vec])
            gt1 = v > l1
            gt2 = v > l2
            l2n = jnp.where(gt1, l1, jnp.where(gt2, v, l2))
            i2n = jnp.where(gt1, i1, jnp.where(gt2, e_vec, i2))
            l1n = jnp.where(gt1, v, l1)
            i1n = jnp.where(gt1, e_vec, i1)
            out.append((l1n, l2n, i1n, i2n))
        return tuple(out)

    init = tuple((neg, neg, zero_i, zero_i) for _ in range(ng))
    return lax.fori_loop(0, _E, body, init)


def _sum_exp_groups(in_tile, rows_list, l1_list):
    ng = len(rows_list)
    zero_f = jnp.zeros((_LANES,), dtype=jnp.float32)

    def body(e, carry):
        e_vec = jnp.full((_LANES,), e, dtype=jnp.int32)
        out = []
        for g in range(ng):
            v = plsc.load_gather(in_tile, [rows_list[g], e_vec])
            out.append(carry[g] + jnp.exp(v - l1_list[g]))
        return tuple(out)

    return lax.fori_loop(0, _E, body, tuple(zero_f for _ in range(ng)))


def _sc_route_body(logits_hbm, zeros_hbm, routed_hbm, idx_hbm,
                   in_t, out_t, idx_t):
    info = plsc.get_sparse_core_info()
    wid = lax.axis_index("s") * info.num_cores + lax.axis_index("c")
    pltpu.sync_copy(zeros_hbm, out_t)

    col0 = jnp.zeros((_LANES,), dtype=jnp.int32)
    col1 = jnp.ones((_LANES,), dtype=jnp.int32)
    zf = jnp.zeros((_LANES,), dtype=jnp.float32)
    one_f = jnp.ones((_LANES,), dtype=jnp.float32)
    eps = jnp.full((_LANES,), 1e-8, dtype=jnp.float32)

    rows_list = [lax.iota(jnp.int32, _LANES) + grp * _LANES
                 for grp in range(_NG)]

    for blk in range(_NBLK):
        base = wid * _ROWS_PER_W + blk * _BLK
        pltpu.sync_copy(logits_hbm.at[pl.ds(base, _BLK)], in_t)
        results = _scan_top2_groups(in_t, rows_list)
        zs = _sum_exp_groups(in_t, rows_list, [r[0] for r in results])
        touched = []
        for g, (l1, l2, i1, i2) in enumerate(results):
            p1 = one_f / zs[g]
            p2 = jnp.exp(l2 - l1) / zs[g]
            denom = p1 + p2 + eps
            plsc.store_scatter(out_t, [rows_list[g], i1], p1 / denom)
            plsc.store_scatter(out_t, [rows_list[g], i2], p2 / denom)
            plsc.store_scatter(idx_t, [rows_list[g], col0], i1)
            plsc.store_scatter(idx_t, [rows_list[g], col1], i2)
            touched.append((rows_list[g], i1, i2))
        pltpu.sync_copy(out_t, routed_hbm.at[pl.ds(base, _BLK)])
        pltpu.sync_copy(idx_t, idx_hbm.at[pl.ds(base, _BLK)])
        for rows, i1, i2 in touched:
            plsc.store_scatter(out_t, [rows, i1], zf)
            plsc.store_scatter(out_t, [rows, i2], zf)


@functools.partial(
    pl.kernel,
    mesh=plsc.VectorSubcoreMesh(core_axis_name="c", subcore_axis_name="s"),
    out_type=[
        jax.ShapeDtypeStruct((_N, _E), jnp.float32),
        jax.ShapeDtypeStruct((_N, 2), jnp.int32),
    ],
    scratch_types=[
        pltpu.VMEM((_BLK, _E), jnp.float32),
        pltpu.VMEM((_BLK, _E), jnp.float32),
        pltpu.VMEM((_BLK, 2), jnp.int32),
    ],
    compiler_params=pltpu.CompilerParams(needs_layout_passes=False),
)
def _sc_route(logits_hbm, zeros_hbm, routed_hbm, idx_hbm, in_t, out_t, idx_t):
    _sc_route_body(logits_hbm, zeros_hbm, routed_hbm, idx_hbm,
                   in_t, out_t, idx_t)


def kernel(fused_latent, ln_g, ln_b, W1, b1, W2, b2):
    W1f = ln_g[:, None] * W1
    b1f = (b1 + ln_b @ W1).reshape(1, _D)
    logits = _tc_logits(fused_latent, W1f, b1f, W2, b2.reshape(1, _E))
    zeros = jnp.zeros((_BLK, _E), dtype=jnp.float32)
    routed, idx = _sc_route(logits, zeros)
    return routed, idx


# speedup vs baseline: 1.1549x; 1.1549x over previous
import functools

import jax
import jax.numpy as jnp
from jax import lax
from jax.experimental import pallas as pl
from jax.experimental.pallas import tpu as pltpu
from jax.experimental.pallas import tpu_sc as plsc

_N = 32768
_D = 768
_E = 64
_BN = 512

_NW = 32
_ROWS_PER_W = _N // _NW
_BLK = 128
_NBLK = _ROWS_PER_W // _BLK
_LANES = 16
_NG = _BLK // _LANES


def _tc_logits_body(x_ref, w1_ref, b1_ref, w2_ref, b2_ref, logits_ref):
    x = x_ref[...]
    mu = jnp.mean(x, axis=-1, keepdims=True)
    xc = x - mu
    var = jnp.mean(xc * xc, axis=-1, keepdims=True)
    xn = xc / jnp.sqrt(var + 1e-5)

    h = jnp.dot(xn, w1_ref[...], preferred_element_type=jnp.float32)
    h = h + b1_ref[...]
    h = 0.5 * h * (1.0 + jax.lax.erf(h * 0.7071067811865476))

    logits = jnp.dot(h, w2_ref[...], preferred_element_type=jnp.float32)
    logits = logits + b2_ref[...]

    m = jnp.max(logits, axis=-1, keepdims=True)
    ex = jnp.exp(logits - m)
    logits_ref[...] = ex / jnp.sum(ex, axis=-1, keepdims=True)


def _tc_logits(x, W1f, b1f, W2, b2f):
    grid = (_N // _BN,)
    return pl.pallas_call(
        _tc_logits_body,
        grid=grid,
        in_specs=[
            pl.BlockSpec((_BN, _D), lambda i: (i, 0)),
            pl.BlockSpec((_D, _D), lambda i: (0, 0)),
            pl.BlockSpec((1, _D), lambda i: (0, 0)),
            pl.BlockSpec((_D, _E), lambda i: (0, 0)),
            pl.BlockSpec((1, _E), lambda i: (0, 0)),
        ],
        out_specs=pl.BlockSpec((_BN, _E), lambda i: (i, 0)),
        out_shape=jax.ShapeDtypeStruct((_N, _E), jnp.float32),
        compiler_params=pltpu.CompilerParams(
            dimension_semantics=("arbitrary",),
        ),
    )(x, W1f, b1f, W2, b2f)


def _scan_top2_groups(in_tile, rows_list):
    ng = len(rows_list)
    neg = jnp.full((_LANES,), -1.0, dtype=jnp.float32)
    zero_i = jnp.zeros((_LANES,), dtype=jnp.int32)

    def body(e, carry):
        e_vec = jnp.full((_LANES,), e, dtype=jnp.int32)
        out = []
        for g in range(ng):
            l1, l2, i1, i2 = carry[g]
            v = plsc.load_gather(in_tile, [rows_list[g], e_vec])
            gt1 = v > l1
            gt2 = v > l2
            l2n = jnp.where(gt1, l1, jnp.where(gt2, v, l2))
            i2n = jnp.where(gt1, i1, jnp.where(gt2, e_vec, i2))
            l1n = jnp.where(gt1, v, l1)
            i1n = jnp.where(gt1, e_vec, i1)
            out.append((l1n, l2n, i1n, i2n))
        return tuple(out)

    init = tuple((neg, neg, zero_i, zero_i) for _ in range(ng))
    return lax.fori_loop(0, _E, body, init)


def _sc_route_body(logits_hbm, zeros_hbm, routed_hbm, idx_hbm,
                   in_t, out_t, idx_t):
    info = plsc.get_sparse_core_info()
    wid = lax.axis_index("s") * info.num_cores + lax.axis_index("c")
    pltpu.sync_copy(zeros_hbm, out_t)

    col0 = jnp.zeros((_LANES,), dtype=jnp.int32)
    col1 = jnp.ones((_LANES,), dtype=jnp.int32)
    zf = jnp.zeros((_LANES,), dtype=jnp.float32)
    eps = jnp.full((_LANES,), 1e-8, dtype=jnp.float32)

    rows_list = [lax.iota(jnp.int32, _LANES) + grp * _LANES
                 for grp in range(_NG)]

    for blk in range(_NBLK):
        base = wid * _ROWS_PER_W + blk * _BLK
        pltpu.sync_copy(logits_hbm.at[pl.ds(base, _BLK)], in_t)
        results = _scan_top2_groups(in_t, rows_list)
        touched = []
        for g, (p1, p2, i1, i2) in enumerate(results):
            denom = p1 + p2 + eps
            plsc.store_scatter(out_t, [rows_list[g], i1], p1 / denom)
            plsc.store_scatter(out_t, [rows_list[g], i2], p2 / denom)
            plsc.store_scatter(idx_t, [rows_list[g], col0], i1)
            plsc.store_scatter(idx_t, [rows_list[g], col1], i2)
            touched.append((rows_list[g], i1, i2))
        pltpu.sync_copy(out_t, routed_hbm.at[pl.ds(base, _BLK)])
        pltpu.sync_copy(idx_t, idx_hbm.at[pl.ds(base, _BLK)])
        for rows, i1, i2 in touched:
            plsc.store_scatter(out_t, [rows, i1], zf)
            plsc.store_scatter(out_t, [rows, i2], zf)


@functools.partial(
    pl.kernel,
    mesh=plsc.VectorSubcoreMesh(core_axis_name="c", subcore_axis_name="s"),
    out_type=[
        jax.ShapeDtypeStruct((_N, _E), jnp.float32),
        jax.ShapeDtypeStruct((_N, 2), jnp.int32),
    ],
    scratch_types=[
        pltpu.VMEM((_BLK, _E), jnp.float32),
        pltpu.VMEM((_BLK, _E), jnp.float32),
        pltpu.VMEM((_BLK, 2), jnp.int32),
    ],
    compiler_params=pltpu.CompilerParams(needs_layout_passes=False),
)
def _sc_route(logits_hbm, zeros_hbm, routed_hbm, idx_hbm, in_t, out_t, idx_t):
    _sc_route_body(logits_hbm, zeros_hbm, routed_hbm, idx_hbm,
                   in_t, out_t, idx_t)


def kernel(fused_latent, ln_g, ln_b, W1, b1, W2, b2):
    W1f = ln_g[:, None] * W1
    b1f = (b1 + ln_b @ W1).reshape(1, _D)
    logits = _tc_logits(fused_latent, W1f, b1f, W2, b2.reshape(1, _E))
    zeros = jnp.zeros((_BLK, _E), dtype=jnp.float32)
    routed, idx = _sc_route(logits, zeros)
    return routed, idx
